# Initial kernel scaffold; baseline (speedup 1.0000x reference)
import jax
import jax.numpy as jnp
from jax.experimental import pallas as pl


def kernel(indices, patterns, amplitude_scale, frequency_shift):
    raise NotImplementedError("write your pallas kernel here")


# speedup vs baseline: 7.4329x; 7.4329x over previous
import functools

import jax
import jax.numpy as jnp
from jax import lax
from jax.experimental import pallas as pl
from jax.experimental.pallas import tpu as pltpu
from jax.experimental.pallas import tpu_sc as plsc

NC = 2
NS = 16
NW = NC * NS
L = 16
D = 256
P = 25
PPAD = 32


def _fold_body(patterns_hbm, scale_hbm, shift_hbm, folded_hbm,
               pat_v, sc_v, sh_v, out_v):
    w = lax.axis_index("s") * NC + lax.axis_index("c")
    pltpu.sync_copy(patterns_hbm, pat_v)
    pltpu.sync_copy(scale_hbm, sc_v)
    pltpu.sync_copy(shift_hbm, sh_v)

    @pl.when(w < P)
    def _():
        wv = jnp.full((L,), w, jnp.int32)
        scale = plsc.load_gather(sc_v, [wv])
        shf = plsc.load_gather(sh_v, [wv])
        s = (shf * 10.0).astype(jnp.int32)
        for c in range(D // L):
            col = lax.iota(jnp.int32, L) + (c * L)
            src = lax.rem(col - s, D)
            src = src + jnp.where(src < 0, D, 0)
            vals = plsc.load_gather(pat_v, [wv, src])
            out_v[pl.ds(c * L, L)] = vals * scale
        pltpu.sync_copy(out_v, folded_hbm.at[w])


def _gather_body(b_per_w, n_chunk, ch,
                 folded_hbm, idx_hbm, out_hbm, idx_c, rows_v, sem):
    w = lax.axis_index("s") * NC + lax.axis_index("c")
    base = w * b_per_w

    def body(c, _):
        off = base + c * ch
        pltpu.sync_copy(idx_hbm.at[pl.ds(off, ch)], idx_c)
        pltpu.async_copy(folded_hbm.at[idx_c], rows_v, sem).wait()
        pltpu.sync_copy(rows_v, out_hbm.at[pl.ds(off, ch)])
        return 0

    lax.fori_loop(0, n_chunk, body, 0)


def kernel(indices, patterns, amplitude_scale, frequency_shift):
    bsz, seq = indices.shape
    n = bsz * seq
    b_per_w = n // NW
    ch = 128
    n_chunk = b_per_w // ch

    mesh = plsc.VectorSubcoreMesh(
        core_axis_name="c", subcore_axis_name="s",
        num_cores=NC, num_subcores=NS)

    scale_p = jnp.zeros((PPAD,), jnp.float32).at[:P].set(amplitude_scale)
    shift_p = jnp.zeros((PPAD,), jnp.float32).at[:P].set(frequency_shift)

    fold = pl.kernel(
        _fold_body,
        out_type=jax.ShapeDtypeStruct((P, D), jnp.float32),
        mesh=mesh,
        compiler_params=pltpu.CompilerParams(needs_layout_passes=False),
        scratch_types=[
            pltpu.VMEM((P, D), jnp.float32),
            pltpu.VMEM((PPAD,), jnp.float32),
            pltpu.VMEM((PPAD,), jnp.float32),
            pltpu.VMEM((D,), jnp.float32),
        ],
    )
    folded = fold(patterns, scale_p, shift_p)

    gather = pl.kernel(
        functools.partial(_gather_body, b_per_w, n_chunk, ch),
        out_type=jax.ShapeDtypeStruct((n, D), jnp.float32),
        mesh=mesh,
        compiler_params=pltpu.CompilerParams(needs_layout_passes=False),
        scratch_types=[
            pltpu.VMEM((ch,), jnp.int32),
            pltpu.VMEM((ch, D), jnp.float32),
            pltpu.SemaphoreType.DMA,
        ],
    )
    out = gather(folded, indices.reshape(n))
    return out.reshape(bsz, seq, D)


# speedup vs baseline: 7.5065x; 1.0099x over previous
import functools

import jax
import jax.numpy as jnp
from jax import lax
from jax.experimental import pallas as pl
from jax.experimental.pallas import tpu as pltpu
from jax.experimental.pallas import tpu_sc as plsc

NC = 2
NS = 16
NW = NC * NS
L = 16
D = 256
P = 25
PPAD = 32


def _fold_body(patterns_hbm, scale_hbm, shift_hbm, folded_hbm,
               pat_v, sc_v, sh_v, out_v):
    w = lax.axis_index("s") * NC + lax.axis_index("c")
    pltpu.sync_copy(patterns_hbm, pat_v)
    pltpu.sync_copy(scale_hbm, sc_v)
    pltpu.sync_copy(shift_hbm, sh_v)

    @pl.when(w < P)
    def _():
        wv = jnp.full((L,), w, jnp.int32)
        scale = plsc.load_gather(sc_v, [wv])
        shf = plsc.load_gather(sh_v, [wv])
        s = (shf * 10.0).astype(jnp.int32)
        for c in range(D // L):
            col = lax.iota(jnp.int32, L) + (c * L)
            src = lax.rem(col - s, D)
            src = src + jnp.where(src < 0, D, 0)
            vals = plsc.load_gather(pat_v, [wv, src])
            out_v[pl.ds(c * L, L)] = vals * scale
        pltpu.sync_copy(out_v, folded_hbm.at[w])


def _gather_body(b_per_w, n_chunk, ch,
                 folded_hbm, idx_hbm, out_hbm,
                 idx_v, rows0, rows1, g0, g1, p0, p1):
    w = lax.axis_index("s") * NC + lax.axis_index("c")
    base = w * b_per_w
    pltpu.sync_copy(idx_hbm.at[pl.ds(base, b_per_w)], idx_v)
    rows = (rows0, rows1)
    gs = (g0, g1)
    ps = (p0, p1)

    def gath(c, b):
        return pltpu.make_async_copy(
            folded_hbm.at[idx_v.at[pl.ds(c * ch, ch)]], rows[b], gs[b])

    def put(c, b):
        return pltpu.make_async_copy(
            rows[b], out_hbm.at[pl.ds(base + c * ch, ch)], ps[b])

    gath(0, 0).start()
    gath(1, 1).start()

    def body(i, _):
        c0 = 2 * i
        for b in range(2):
            gath(c0 + b, b).wait()
            put(c0 + b, b).start()
        for b in range(2):
            put(c0 + b, b).wait()

            @pl.when(c0 + b + 2 < n_chunk)
            def _():
                gath(c0 + b + 2, b).start()
        return 0

    lax.fori_loop(0, n_chunk // 2, body, 0)


def kernel(indices, patterns, amplitude_scale, frequency_shift):
    bsz, seq = indices.shape
    n = bsz * seq
    b_per_w = n // NW
    ch = 128
    n_chunk = b_per_w // ch

    mesh = plsc.VectorSubcoreMesh(
        core_axis_name="c", subcore_axis_name="s",
        num_cores=NC, num_subcores=NS)

    scale_p = jnp.zeros((PPAD,), jnp.float32).at[:P].set(amplitude_scale)
    shift_p = jnp.zeros((PPAD,), jnp.float32).at[:P].set(frequency_shift)

    fold = pl.kernel(
        _fold_body,
        out_type=jax.ShapeDtypeStruct((P, D), jnp.float32),
        mesh=mesh,
        compiler_params=pltpu.CompilerParams(needs_layout_passes=False),
        scratch_types=[
            pltpu.VMEM((P, D), jnp.float32),
            pltpu.VMEM((PPAD,), jnp.float32),
            pltpu.VMEM((PPAD,), jnp.float32),
            pltpu.VMEM((D,), jnp.float32),
        ],
    )
    folded = fold(patterns, scale_p, shift_p)

    gather = pl.kernel(
        functools.partial(_gather_body, b_per_w, n_chunk, ch),
        out_type=jax.ShapeDtypeStruct((n, D), jnp.float32),
        mesh=mesh,
        compiler_params=pltpu.CompilerParams(needs_layout_passes=False),
        scratch_types=[
            pltpu.VMEM((b_per_w,), jnp.int32),
            pltpu.VMEM((ch, D), jnp.float32),
            pltpu.VMEM((ch, D), jnp.float32),
            pltpu.SemaphoreType.DMA,
            pltpu.SemaphoreType.DMA,
            pltpu.SemaphoreType.DMA,
            pltpu.SemaphoreType.DMA,
        ],
    )
    out = gather(folded, indices.reshape(n))
    return out.reshape(bsz, seq, D)


# speedup vs baseline: 15.7962x; 2.1043x over previous
import functools

import jax
import jax.numpy as jnp
from jax import lax
from jax.experimental import pallas as pl
from jax.experimental.pallas import tpu as pltpu
from jax.experimental.pallas import tpu_sc as plsc

NC = 2
NS = 16
NW = NC * NS
L = 16
D = 256
P = 25
PPAD = 32
K = 16


def _fold_body(patterns_hbm, scale_hbm, shift_hbm, folded_hbm,
               pat_v, sc_v, sh_v, out_v):
    w = lax.axis_index("s") * NC + lax.axis_index("c")
    pltpu.sync_copy(patterns_hbm, pat_v)
    pltpu.sync_copy(scale_hbm, sc_v)
    pltpu.sync_copy(shift_hbm, sh_v)

    @pl.when(w < P)
    def _():
        wv = jnp.full((L,), w, jnp.int32)
        scale = plsc.load_gather(sc_v, [wv])
        shf = plsc.load_gather(sh_v, [wv])
        s = (shf * 10.0).astype(jnp.int32)
        for c in range(D // L):
            col = lax.iota(jnp.int32, L) + (c * L)
            src = lax.rem(col - s, D)
            src = src + jnp.where(src < 0, D, 0)
            vals = plsc.load_gather(pat_v, [wv, src])
            out_v[pl.ds(c * L, L)] = vals * scale
        for k in range(K):
            pltpu.sync_copy(out_v, folded_hbm.at[k * P + w])


def _gather_body(b_per_w, n_chunk, ch,
                 folded_hbm, idx_hbm, out_hbm,
                 idx_v, rows0, rows1, g0, g1, p0, p1):
    w = lax.axis_index("s") * NC + lax.axis_index("c")
    base = w * b_per_w
    pltpu.sync_copy(idx_hbm.at[pl.ds(base, b_per_w)], idx_v)

    offs = (lax.iota(jnp.int32, L) % K) * P

    def spread(j, _):
        idx_v[pl.ds(j * L, L)] = idx_v[pl.ds(j * L, L)] + offs
        return 0

    lax.fori_loop(0, b_per_w // L, spread, 0)
    rows = (rows0, rows1)
    gs = (g0, g1)
    ps = (p0, p1)

    def gath(c, b):
        return pltpu.make_async_copy(
            folded_hbm.at[idx_v.at[pl.ds(c * ch, ch)]], rows[b], gs[b])

    def put(c, b):
        return pltpu.make_async_copy(
            rows[b], out_hbm.at[pl.ds(base + c * ch, ch)], ps[b])

    gath(0, 0).start()
    gath(1, 1).start()

    def body(i, _):
        c0 = 2 * i
        for b in range(2):
            gath(c0 + b, b).wait()
            put(c0 + b, b).start()
        for b in range(2):
            put(c0 + b, b).wait()

            @pl.when(c0 + b + 2 < n_chunk)
            def _():
                gath(c0 + b + 2, b).start()
        return 0

    lax.fori_loop(0, n_chunk // 2, body, 0)


def kernel(indices, patterns, amplitude_scale, frequency_shift):
    bsz, seq = indices.shape
    n = bsz * seq
    b_per_w = n // NW
    ch = 128
    n_chunk = b_per_w // ch

    mesh = plsc.VectorSubcoreMesh(
        core_axis_name="c", subcore_axis_name="s",
        num_cores=NC, num_subcores=NS)

    scale_p = jnp.zeros((PPAD,), jnp.float32).at[:P].set(amplitude_scale)
    shift_p = jnp.zeros((PPAD,), jnp.float32).at[:P].set(frequency_shift)

    fold = pl.kernel(
        _fold_body,
        out_type=jax.ShapeDtypeStruct((K * P, D), jnp.float32),
        mesh=mesh,
        compiler_params=pltpu.CompilerParams(needs_layout_passes=False),
        scratch_types=[
            pltpu.VMEM((P, D), jnp.float32),
            pltpu.VMEM((PPAD,), jnp.float32),
            pltpu.VMEM((PPAD,), jnp.float32),
            pltpu.VMEM((D,), jnp.float32),
        ],
    )
    folded = fold(patterns, scale_p, shift_p)

    gather = pl.kernel(
        functools.partial(_gather_body, b_per_w, n_chunk, ch),
        out_type=jax.ShapeDtypeStruct((n, D), jnp.float32),
        mesh=mesh,
        compiler_params=pltpu.CompilerParams(needs_layout_passes=False),
        scratch_types=[
            pltpu.VMEM((b_per_w,), jnp.int32),
            pltpu.VMEM((ch, D), jnp.float32),
            pltpu.VMEM((ch, D), jnp.float32),
            pltpu.SemaphoreType.DMA,
            pltpu.SemaphoreType.DMA,
            pltpu.SemaphoreType.DMA,
            pltpu.SemaphoreType.DMA,
        ],
    )
    out = gather(folded, indices.reshape(n))
    return out.reshape(bsz, seq, D)


# speedup vs baseline: 20.7451x; 1.3133x over previous
import functools

import jax
import jax.numpy as jnp
from jax import lax
from jax.experimental import pallas as pl
from jax.experimental.pallas import tpu as pltpu
from jax.experimental.pallas import tpu_sc as plsc

NC = 2
NS = 16
NW = NC * NS
L = 16
D = 256
P = 25
PPAD = 32
K = 32


def _fold_body(patterns_hbm, scale_hbm, shift_hbm, folded_hbm,
               pat_v, sc_v, sh_v, out_v):
    w = lax.axis_index("s") * NC + lax.axis_index("c")
    pltpu.sync_copy(patterns_hbm, pat_v)
    pltpu.sync_copy(scale_hbm, sc_v)
    pltpu.sync_copy(shift_hbm, sh_v)

    @pl.when(w < P)
    def _():
        wv = jnp.full((L,), w, jnp.int32)
        scale = plsc.load_gather(sc_v, [wv])
        shf = plsc.load_gather(sh_v, [wv])
        s = (shf * 10.0).astype(jnp.int32)
        for c in range(D // L):
            col = lax.iota(jnp.int32, L) + (c * L)
            src = lax.rem(col - s, D)
            src = src + jnp.where(src < 0, D, 0)
            vals = plsc.load_gather(pat_v, [wv, src])
            out_v[pl.ds(c * L, L)] = vals * scale
        for k in range(K):
            pltpu.sync_copy(out_v, folded_hbm.at[k * P + w])


def _gather_body(b_per_w, n_chunk, ch,
                 folded_hbm, idx_hbm, out_hbm,
                 idx_v, rows0, rows1, g0, g1, p0, p1):
    w = lax.axis_index("s") * NC + lax.axis_index("c")
    base = w * b_per_w
    pltpu.sync_copy(idx_hbm.at[pl.ds(base, b_per_w)], idx_v)

    def spread(j, _):
        offs = ((lax.iota(jnp.int32, L) + j * L) % K) * P
        idx_v[pl.ds(j * L, L)] = idx_v[pl.ds(j * L, L)] + offs
        return 0

    lax.fori_loop(0, b_per_w // L, spread, 0)
    rows = (rows0, rows1)
    gs = (g0, g1)
    ps = (p0, p1)

    def gath(c, b):
        return pltpu.make_async_copy(
            folded_hbm.at[idx_v.at[pl.ds(c * ch, ch)]], rows[b], gs[b])

    def put(c, b):
        return pltpu.make_async_copy(
            rows[b], out_hbm.at[pl.ds(base + c * ch, ch)], ps[b])

    gath(0, 0).start()
    gath(1, 1).start()

    def body(i, _):
        c0 = 2 * i
        for b in range(2):
            gath(c0 + b, b).wait()
            put(c0 + b, b).start()
        for b in range(2):
            put(c0 + b, b).wait()

            @pl.when(c0 + b + 2 < n_chunk)
            def _():
                gath(c0 + b + 2, b).start()
        return 0

    lax.fori_loop(0, n_chunk // 2, body, 0)


def kernel(indices, patterns, amplitude_scale, frequency_shift):
    bsz, seq = indices.shape
    n = bsz * seq
    b_per_w = n // NW
    ch = 128
    n_chunk = b_per_w // ch

    mesh = plsc.VectorSubcoreMesh(
        core_axis_name="c", subcore_axis_name="s",
        num_cores=NC, num_subcores=NS)

    scale_p = jnp.zeros((PPAD,), jnp.float32).at[:P].set(amplitude_scale)
    shift_p = jnp.zeros((PPAD,), jnp.float32).at[:P].set(frequency_shift)

    fold = pl.kernel(
        _fold_body,
        out_type=jax.ShapeDtypeStruct((K * P, D), jnp.float32),
        mesh=mesh,
        compiler_params=pltpu.CompilerParams(needs_layout_passes=False),
        scratch_types=[
            pltpu.VMEM((P, D), jnp.float32),
            pltpu.VMEM((PPAD,), jnp.float32),
            pltpu.VMEM((PPAD,), jnp.float32),
            pltpu.VMEM((D,), jnp.float32),
        ],
    )
    folded = fold(patterns, scale_p, shift_p)

    gather = pl.kernel(
        functools.partial(_gather_body, b_per_w, n_chunk, ch),
        out_type=jax.ShapeDtypeStruct((n, D), jnp.float32),
        mesh=mesh,
        compiler_params=pltpu.CompilerParams(needs_layout_passes=False),
        scratch_types=[
            pltpu.VMEM((b_per_w,), jnp.int32),
            pltpu.VMEM((ch, D), jnp.float32),
            pltpu.VMEM((ch, D), jnp.float32),
            pltpu.SemaphoreType.DMA,
            pltpu.SemaphoreType.DMA,
            pltpu.SemaphoreType.DMA,
            pltpu.SemaphoreType.DMA,
        ],
    )
    out = gather(folded, indices.reshape(n))
    return out.reshape(bsz, seq, D)


# speedup vs baseline: 22.0513x; 1.0630x over previous
import functools

import jax
import jax.numpy as jnp
from jax import lax
from jax.experimental import pallas as pl
from jax.experimental.pallas import tpu as pltpu
from jax.experimental.pallas import tpu_sc as plsc

NC = 2
NS = 16
NW = NC * NS
L = 16
D = 256
P = 25
PPAD = 32
K = 64


def _fold_body(patterns_hbm, scale_hbm, shift_hbm, folded_hbm,
               pat_v, sc_v, sh_v, out_v):
    w = lax.axis_index("s") * NC + lax.axis_index("c")
    pltpu.sync_copy(patterns_hbm, pat_v)
    pltpu.sync_copy(scale_hbm, sc_v)
    pltpu.sync_copy(shift_hbm, sh_v)

    @pl.when(w < P)
    def _():
        wv = jnp.full((L,), w, jnp.int32)
        scale = plsc.load_gather(sc_v, [wv])
        shf = plsc.load_gather(sh_v, [wv])
        s = (shf * 10.0).astype(jnp.int32)
        for c in range(D // L):
            col = lax.iota(jnp.int32, L) + (c * L)
            src = lax.rem(col - s, D)
            src = src + jnp.where(src < 0, D, 0)
            vals = plsc.load_gather(pat_v, [wv, src])
            out_v[pl.ds(c * L, L)] = vals * scale
        for k in range(K):
            pltpu.sync_copy(out_v, folded_hbm.at[k * P + w])


def _gather_body(b_per_w, n_chunk, ch,
                 folded_hbm, idx_hbm, out_hbm,
                 idx_v, rows0, rows1, g0, g1, p0, p1):
    w = lax.axis_index("s") * NC + lax.axis_index("c")
    base = w * b_per_w
    pltpu.sync_copy(idx_hbm.at[pl.ds(base, b_per_w)], idx_v)

    def spread(j, _):
        offs = ((lax.iota(jnp.int32, L) + j * L) % K) * P
        idx_v[pl.ds(j * L, L)] = idx_v[pl.ds(j * L, L)] + offs
        return 0

    lax.fori_loop(0, b_per_w // L, spread, 0)
    rows = (rows0, rows1)
    gs = (g0, g1)
    ps = (p0, p1)

    def gath(c, b):
        return pltpu.make_async_copy(
            folded_hbm.at[idx_v.at[pl.ds(c * ch, ch)]], rows[b], gs[b])

    def put(c, b):
        return pltpu.make_async_copy(
            rows[b], out_hbm.at[pl.ds(base + c * ch, ch)], ps[b])

    gath(0, 0).start()
    gath(1, 1).start()

    def body(i, _):
        c0 = 2 * i
        for b in range(2):
            gath(c0 + b, b).wait()
            put(c0 + b, b).start()
        for b in range(2):
            put(c0 + b, b).wait()

            @pl.when(c0 + b + 2 < n_chunk)
            def _():
                gath(c0 + b + 2, b).start()
        return 0

    lax.fori_loop(0, n_chunk // 2, body, 0)


def kernel(indices, patterns, amplitude_scale, frequency_shift):
    bsz, seq = indices.shape
    n = bsz * seq
    b_per_w = n // NW
    ch = 128
    n_chunk = b_per_w // ch

    mesh = plsc.VectorSubcoreMesh(
        core_axis_name="c", subcore_axis_name="s",
        num_cores=NC, num_subcores=NS)

    scale_p = jnp.zeros((PPAD,), jnp.float32).at[:P].set(amplitude_scale)
    shift_p = jnp.zeros((PPAD,), jnp.float32).at[:P].set(frequency_shift)

    fold = pl.kernel(
        _fold_body,
        out_type=jax.ShapeDtypeStruct((K * P, D), jnp.float32),
        mesh=mesh,
        compiler_params=pltpu.CompilerParams(needs_layout_passes=False),
        scratch_types=[
            pltpu.VMEM((P, D), jnp.float32),
            pltpu.VMEM((PPAD,), jnp.float32),
            pltpu.VMEM((PPAD,), jnp.float32),
            pltpu.VMEM((D,), jnp.float32),
        ],
    )
    folded = fold(patterns, scale_p, shift_p)

    gather = pl.kernel(
        functools.partial(_gather_body, b_per_w, n_chunk, ch),
        out_type=jax.ShapeDtypeStruct((n, D), jnp.float32),
        mesh=mesh,
        compiler_params=pltpu.CompilerParams(needs_layout_passes=False),
        scratch_types=[
            pltpu.VMEM((b_per_w,), jnp.int32),
            pltpu.VMEM((ch, D), jnp.float32),
            pltpu.VMEM((ch, D), jnp.float32),
            pltpu.SemaphoreType.DMA,
            pltpu.SemaphoreType.DMA,
            pltpu.SemaphoreType.DMA,
            pltpu.SemaphoreType.DMA,
        ],
    )
    out = gather(folded, indices.reshape(n))
    return out.reshape(bsz, seq, D)
